# baseline (speedup 1.0000x reference)
import jax
import jax.numpy as jnp
from jax import lax
from jax.experimental import pallas as pl
from jax.experimental.pallas import tpu as pltpu
from jax.experimental.pallas import tpu_sc as plsc

N = 10000
E = 320000
D = 128
NC = 2
NS = 16

EPC = E // NC
EPT = EPC // NS
G = 128
NG = EPT // G
REM = EPT - NG * G
NP = NG // 2

EPW = E // (NC * NS)
DCHUNK = 400
NDCHUNK = EPW // DCHUNK

ROWS_PT = 624
TAIL = N - ROWS_PT * NS


def _sc_hist_body(dst_h, hists_h, ibuf, hist):
    cid = lax.axis_index("c")
    sid = lax.axis_index("s")
    zeros16 = jnp.zeros((16,), jnp.float32)

    def zh(i, c):
        hist[pl.ds(i * 16, 16)] = zeros16
        return c

    lax.fori_loop(0, N // 16, zh, 0)
    wid = cid * NS + sid
    dbase = wid * EPW
    ones16 = jnp.full((16,), 1.0, jnp.float32)

    def dchunk(q, c):
        pltpu.sync_copy(dst_h.at[pl.ds(dbase + q * DCHUNK, DCHUNK)], ibuf)
        for j in range(DCHUNK // 16):
            v = ibuf[pl.ds(j * 16, 16)]
            plsc.addupdate_scatter(hist, [v], ones16)
        return c

    lax.fori_loop(0, NDCHUNK, dchunk, 0)
    pltpu.sync_copy(hist, hists_h.at[wid, 0])


_sc_hist = pl.kernel(
    _sc_hist_body,
    out_type=(jax.ShapeDtypeStruct((NC * NS, 1, N), jnp.float32),),
    mesh=plsc.VectorSubcoreMesh(core_axis_name="c", subcore_axis_name="s"),
    scratch_types=[
        pltpu.VMEM((DCHUNK,), jnp.int32),
        pltpu.VMEM((N,), jnp.float32),
    ],
    compiler_params=pltpu.CompilerParams(needs_layout_passes=False),
)


def _sc_agg_body(tbl_h, src_h, dst_h, agg_h,
                 agg_sp, srcA, srcB, dstA, dstB, srcr, dstr, rows0, rows1,
                 isem, gsem, ssem):
    cid = lax.axis_index("c")
    sid = lax.axis_index("s")

    zeros16 = jnp.zeros((16,), jnp.float32)

    def zrow(i, c):
        for j in range(D // 16):
            rows0[i, pl.ds(j * 16, 16)] = zeros16
        return c

    lax.fori_loop(0, G, zrow, 0)
    zb = sid * ROWS_PT
    for k in range(4):
        pltpu.sync_copy(rows0, agg_sp.at[pl.ds(zb + k * 128, 128)])
    pltpu.sync_copy(rows0.at[pl.ds(0, ROWS_PT - 512)],
                    agg_sp.at[pl.ds(zb + 512, ROWS_PT - 512)])

    @pl.when(sid == 0)
    def _():
        pltpu.sync_copy(rows0.at[pl.ds(0, TAIL)],
                        agg_sp.at[pl.ds(ROWS_PT * NS, TAIL)])

    plsc.subcore_barrier()

    tile_base = cid * EPC + sid * EPT

    def idx_load(g, sbuf, dbuf):
        base = tile_base + g * G
        a = pltpu.async_copy(src_h.at[pl.ds(base, G)], sbuf, isem)
        b = pltpu.async_copy(dst_h.at[pl.ds(base, G)], dbuf, isem)
        return a, b

    def pair(p, c):
        ia0, ib0 = idx_load(2 * p, srcA, dstA)
        ia1, ib1 = idx_load(2 * p + 1, srcB, dstB)
        ia0.wait()
        ib0.wait()
        g0 = pltpu.async_copy(tbl_h.at[srcA], rows0, gsem)
        ia1.wait()
        ib1.wait()
        g1 = pltpu.async_copy(tbl_h.at[srcB], rows1, gsem)
        g0.wait()
        s0 = pltpu.async_copy(rows0, agg_sp.at[dstA], ssem, add=True)
        g1.wait()
        s1 = pltpu.async_copy(rows1, agg_sp.at[dstB], ssem, add=True)
        s0.wait()
        s1.wait()
        return c

    lax.fori_loop(0, NP, pair, 0)

    rbase = tile_base + NG * G
    ra = pltpu.async_copy(src_h.at[pl.ds(rbase, REM)], srcr, isem)
    rb = pltpu.async_copy(dst_h.at[pl.ds(rbase, REM)], dstr, isem)
    ra.wait()
    rb.wait()
    pltpu.async_copy(tbl_h.at[srcr], rows0.at[pl.ds(0, REM)], gsem).wait()
    pltpu.async_copy(rows0.at[pl.ds(0, REM)], agg_sp.at[dstr], ssem,
                     add=True).wait()

    plsc.subcore_barrier()

    ob = sid * ROWS_PT
    pltpu.sync_copy(agg_sp.at[pl.ds(ob, ROWS_PT)],
                    agg_h.at[cid, pl.ds(ob, ROWS_PT)])

    @pl.when(sid == 0)
    def _():
        pltpu.sync_copy(agg_sp.at[pl.ds(ROWS_PT * NS, TAIL)],
                        agg_h.at[cid, pl.ds(ROWS_PT * NS, TAIL)])


_sc_agg = pl.kernel(
    _sc_agg_body,
    out_type=(jax.ShapeDtypeStruct((NC, N, D), jnp.float32),),
    mesh=plsc.VectorSubcoreMesh(core_axis_name="c", subcore_axis_name="s"),
    scratch_types=[
        pltpu.VMEM_SHARED((N, D), jnp.float32),
        pltpu.VMEM((G,), jnp.int32),
        pltpu.VMEM((G,), jnp.int32),
        pltpu.VMEM((G,), jnp.int32),
        pltpu.VMEM((G,), jnp.int32),
        pltpu.VMEM((REM,), jnp.int32),
        pltpu.VMEM((REM,), jnp.int32),
        pltpu.VMEM((G, D), jnp.float32),
        pltpu.VMEM((G, D), jnp.float32),
        pltpu.SemaphoreType.DMA,
        pltpu.SemaphoreType.DMA,
        pltpu.SemaphoreType.DMA,
    ],
    compiler_params=pltpu.CompilerParams(needs_layout_passes=False),
)


BLK = 1000
NBLK = N // BLK


def _tc_project_body(x_ref, w_ref, out_ref):
    out_ref[...] = jnp.dot(x_ref[...], w_ref[...],
                           preferred_element_type=jnp.float32)


def _tc_project(x, w):
    return pl.pallas_call(
        _tc_project_body,
        grid=(NBLK,),
        in_specs=[pl.BlockSpec((BLK, D), lambda i: (i, 0)),
                  pl.BlockSpec((D, D), lambda i: (0, 0))],
        out_specs=pl.BlockSpec((BLK, D), lambda i: (i, 0)),
        out_shape=jax.ShapeDtypeStruct((N, D), jnp.float32),
    )(x, w)


def _inv_deg(hists):
    deg = jnp.sum(hists, axis=0)
    return 1.0 / jnp.maximum(deg, 1.0)


def _tc_mid_body(agg_ref, h_ref, x_ref, wr0_ref, b0_ref, wl1_ref, wr1_ref,
                 p1_ref, r1_ref):
    inv = _inv_deg(h_ref[...])
    agg = agg_ref[0] + agg_ref[1]
    mean = agg * inv
    h = jnp.maximum(
        mean + b0_ref[...]
        + jnp.dot(x_ref[...], wr0_ref[...], preferred_element_type=jnp.float32),
        0.0)
    p1_ref[...] = jnp.dot(h, wl1_ref[...], preferred_element_type=jnp.float32)
    r1_ref[...] = jnp.dot(h, wr1_ref[...], preferred_element_type=jnp.float32)


def _tc_mid(agg0, hists, x, wr0, b0, wl1, wr1):
    return pl.pallas_call(
        _tc_mid_body,
        grid=(NBLK,),
        in_specs=[pl.BlockSpec((NC, BLK, D), lambda i: (0, i, 0)),
                  pl.BlockSpec((NC * NS, BLK, 1), lambda i: (0, i, 0)),
                  pl.BlockSpec((BLK, D), lambda i: (i, 0)),
                  pl.BlockSpec((D, D), lambda i: (0, 0)),
                  pl.BlockSpec((1, D), lambda i: (0, 0)),
                  pl.BlockSpec((D, D), lambda i: (0, 0)),
                  pl.BlockSpec((D, D), lambda i: (0, 0))],
        out_specs=[pl.BlockSpec((BLK, D), lambda i: (i, 0)),
                   pl.BlockSpec((BLK, D), lambda i: (i, 0))],
        out_shape=[jax.ShapeDtypeStruct((N, D), jnp.float32),
                   jax.ShapeDtypeStruct((N, D), jnp.float32)],
    )(agg0, hists, x, wr0, b0, wl1, wr1)


DO = 64


def _tc_final_body(agg_ref, h_ref, r1_ref, b1_ref, wlin_ref, blin_ref,
                   out_ref):
    inv = _inv_deg(h_ref[...])
    agg = agg_ref[0] + agg_ref[1]
    h2 = jnp.maximum(agg * inv + b1_ref[...] + r1_ref[...], 0.0)
    o = jnp.dot(h2, wlin_ref[...], preferred_element_type=jnp.float32)
    o = o + blin_ref[...]
    m = jnp.max(o, axis=1, keepdims=True)
    e = jnp.exp(o - m)
    out_ref[...] = e / jnp.sum(e, axis=1, keepdims=True)


def _tc_final(agg1, hists, r1, b1, wlin, blin):
    return pl.pallas_call(
        _tc_final_body,
        grid=(NBLK,),
        in_specs=[pl.BlockSpec((NC, BLK, D), lambda i: (0, i, 0)),
                  pl.BlockSpec((NC * NS, BLK, 1), lambda i: (0, i, 0)),
                  pl.BlockSpec((BLK, D), lambda i: (i, 0)),
                  pl.BlockSpec((1, D), lambda i: (0, 0)),
                  pl.BlockSpec((D, DO), lambda i: (0, 0)),
                  pl.BlockSpec((1, DO), lambda i: (0, 0))],
        out_specs=pl.BlockSpec((BLK, DO), lambda i: (i, 0)),
        out_shape=jax.ShapeDtypeStruct((N, DO), jnp.float32),
    )(agg1, hists, r1, b1, wlin, blin)


def kernel(x, edge_index, Wl0, b0, Wr0, Wl1, b1, Wr1, Wlin, blin):
    src = edge_index[0]
    dst = edge_index[1]

    hists, = _sc_hist(dst)
    hists = hists.reshape(NC * NS, N, 1)
    p0 = _tc_project(x, Wl0)
    agg0, = _sc_agg(p0, src, dst)
    p1, r1 = _tc_mid(agg0, hists, x, Wr0, b0.reshape(1, D), Wl1, Wr1)
    agg1, = _sc_agg(p1, src, dst)
    out = _tc_final(agg1, hists, r1, b1.reshape(1, D), Wlin,
                    blin.reshape(1, DO))
    return out


# speedup vs baseline: 1.1519x; 1.1519x over previous
import functools

import jax
import jax.numpy as jnp
from jax import lax
from jax.experimental import pallas as pl
from jax.experimental.pallas import tpu as pltpu
from jax.experimental.pallas import tpu_sc as plsc

N = 10000
E = 320000
D = 128
NC = 2
NS = 16

EPC = E // NC
EPT = EPC // NS
G = 128
NG = EPT // G
REM = EPT - NG * G

EPW = E // (NC * NS)
DCHUNK = 400
NDCHUNK = EPW // DCHUNK

ROWS_PT = 624
TAIL = N - ROWS_PT * NS


def _sc_agg_body(with_hist, nbuf, *refs):
    it = iter(refs)
    tbl_h = next(it)
    src_h = next(it)
    dst_h = next(it)
    agg_h = next(it)
    hists_h = next(it) if with_hist else None
    agg_sp = next(it)
    srcs = [next(it) for _ in range(nbuf)]
    dsts = [next(it) for _ in range(nbuf)]
    srcr = next(it)
    dstr = next(it)
    rows = [next(it) for _ in range(nbuf)]
    hist = next(it) if with_hist else None
    hbuf = next(it) if with_hist else None
    isem = next(it)
    gsem = next(it)
    ssem = next(it)

    cid = lax.axis_index("c")
    sid = lax.axis_index("s")

    zeros16 = jnp.zeros((16,), jnp.float32)

    def zrow(i, c):
        for j in range(D // 16):
            rows[0][i, pl.ds(j * 16, 16)] = zeros16
        return c

    lax.fori_loop(0, G, zrow, 0)
    zb = sid * ROWS_PT
    for k in range(4):
        pltpu.sync_copy(rows[0], agg_sp.at[pl.ds(zb + k * 128, 128)])
    pltpu.sync_copy(rows[0].at[pl.ds(0, ROWS_PT - 512)],
                    agg_sp.at[pl.ds(zb + 512, ROWS_PT - 512)])

    @pl.when(sid == 0)
    def _():
        pltpu.sync_copy(rows[0].at[pl.ds(0, TAIL)],
                        agg_sp.at[pl.ds(ROWS_PT * NS, TAIL)])

    if with_hist:
        def zh(i, c):
            hist[pl.ds(i * 16, 16)] = zeros16
            return c

        lax.fori_loop(0, N // 16, zh, 0)
        wid = cid * NS + sid
        dbase = wid * EPW
        ones16 = jnp.full((16,), 1.0, jnp.float32)

        def dchunk(q, c):
            pltpu.sync_copy(dst_h.at[pl.ds(dbase + q * DCHUNK, DCHUNK)],
                            hbuf)
            for j in range(DCHUNK // 16):
                v = hbuf[pl.ds(j * 16, 16)]
                plsc.addupdate_scatter(hist, [v], ones16)
            return c

        lax.fori_loop(0, NDCHUNK, dchunk, 0)
        pltpu.sync_copy(hist, hists_h.at[wid, 0])

    plsc.subcore_barrier()

    tile_base = cid * EPC + sid * EPT

    def chunk(q, c):
        g0 = q * nbuf
        ids = []
        for i in range(nbuf):
            base = tile_base + (g0 + i) * G
            a = pltpu.async_copy(src_h.at[pl.ds(base, G)], srcs[i], isem)
            b = pltpu.async_copy(dst_h.at[pl.ds(base, G)], dsts[i], isem)
            ids.append((a, b))
        gds = []
        for i in range(nbuf):
            ids[i][0].wait()
            ids[i][1].wait()
            gds.append(pltpu.async_copy(tbl_h.at[srcs[i]], rows[i], gsem))
        sds = []
        for i in range(nbuf):
            gds[i].wait()
            sds.append(pltpu.async_copy(rows[i], agg_sp.at[dsts[i]], ssem,
                                        add=True))
        for i in range(nbuf):
            sds[i].wait()
        return c

    lax.fori_loop(0, NG // nbuf, chunk, 0)

    rbase = tile_base + NG * G
    ra = pltpu.async_copy(src_h.at[pl.ds(rbase, REM)], srcr, isem)
    rb = pltpu.async_copy(dst_h.at[pl.ds(rbase, REM)], dstr, isem)
    ra.wait()
    rb.wait()
    pltpu.async_copy(tbl_h.at[srcr], rows[0].at[pl.ds(0, REM)], gsem).wait()
    pltpu.async_copy(rows[0].at[pl.ds(0, REM)], agg_sp.at[dstr], ssem,
                     add=True).wait()

    plsc.subcore_barrier()

    ob = sid * ROWS_PT
    pltpu.sync_copy(agg_sp.at[pl.ds(ob, ROWS_PT)],
                    agg_h.at[cid, pl.ds(ob, ROWS_PT)])

    @pl.when(sid == 0)
    def _():
        pltpu.sync_copy(agg_sp.at[pl.ds(ROWS_PT * NS, TAIL)],
                        agg_h.at[cid, pl.ds(ROWS_PT * NS, TAIL)])


def _make_sc_agg(with_hist, nbuf):
    assert NG % nbuf == 0
    out_type = [jax.ShapeDtypeStruct((NC, N, D), jnp.float32)]
    if with_hist:
        out_type.append(jax.ShapeDtypeStruct((NC * NS, 1, N), jnp.float32))
    scratch = [pltpu.VMEM_SHARED((N, D), jnp.float32)]
    scratch += [pltpu.VMEM((G,), jnp.int32) for _ in range(2 * nbuf)]
    scratch += [pltpu.VMEM((REM,), jnp.int32)] * 2
    scratch += [pltpu.VMEM((G, D), jnp.float32) for _ in range(nbuf)]
    if with_hist:
        scratch.append(pltpu.VMEM((N,), jnp.float32))
        scratch.append(pltpu.VMEM((DCHUNK,), jnp.int32))
    scratch += [pltpu.SemaphoreType.DMA] * 3
    return pl.kernel(
        functools.partial(_sc_agg_body, with_hist, nbuf),
        out_type=tuple(out_type),
        mesh=plsc.VectorSubcoreMesh(core_axis_name="c", subcore_axis_name="s"),
        scratch_types=scratch,
        compiler_params=pltpu.CompilerParams(needs_layout_passes=False),
    )


BLK = 1000
NBLK = N // BLK


def _tc_project_body(x_ref, w_ref, out_ref):
    out_ref[...] = jnp.dot(x_ref[...], w_ref[...],
                           preferred_element_type=jnp.float32)


def _tc_project(x, w):
    return pl.pallas_call(
        _tc_project_body,
        grid=(NBLK,),
        in_specs=[pl.BlockSpec((BLK, D), lambda i: (i, 0)),
                  pl.BlockSpec((D, D), lambda i: (0, 0))],
        out_specs=pl.BlockSpec((BLK, D), lambda i: (i, 0)),
        out_shape=jax.ShapeDtypeStruct((N, D), jnp.float32),
    )(x, w)


def _inv_deg(hists):
    deg = jnp.sum(hists, axis=0)
    return 1.0 / jnp.maximum(deg, 1.0)


def _tc_mid_body(agg_ref, h_ref, x_ref, wr0_ref, b0_ref, wl1_ref, wr1_ref,
                 p1_ref, r1_ref):
    inv = _inv_deg(h_ref[...])
    agg = agg_ref[0] + agg_ref[1]
    mean = agg * inv
    h = jnp.maximum(
        mean + b0_ref[...]
        + jnp.dot(x_ref[...], wr0_ref[...], preferred_element_type=jnp.float32),
        0.0)
    p1_ref[...] = jnp.dot(h, wl1_ref[...], preferred_element_type=jnp.float32)
    r1_ref[...] = jnp.dot(h, wr1_ref[...], preferred_element_type=jnp.float32)


def _tc_mid(agg0, hists, x, wr0, b0, wl1, wr1):
    return pl.pallas_call(
        _tc_mid_body,
        grid=(NBLK,),
        in_specs=[pl.BlockSpec((NC, BLK, D), lambda i: (0, i, 0)),
                  pl.BlockSpec((NC * NS, BLK, 1), lambda i: (0, i, 0)),
                  pl.BlockSpec((BLK, D), lambda i: (i, 0)),
                  pl.BlockSpec((D, D), lambda i: (0, 0)),
                  pl.BlockSpec((1, D), lambda i: (0, 0)),
                  pl.BlockSpec((D, D), lambda i: (0, 0)),
                  pl.BlockSpec((D, D), lambda i: (0, 0))],
        out_specs=[pl.BlockSpec((BLK, D), lambda i: (i, 0)),
                   pl.BlockSpec((BLK, D), lambda i: (i, 0))],
        out_shape=[jax.ShapeDtypeStruct((N, D), jnp.float32),
                   jax.ShapeDtypeStruct((N, D), jnp.float32)],
    )(agg0, hists, x, wr0, b0, wl1, wr1)


DO = 64


def _tc_final_body(agg_ref, h_ref, r1_ref, b1_ref, wlin_ref, blin_ref,
                   out_ref):
    inv = _inv_deg(h_ref[...])
    agg = agg_ref[0] + agg_ref[1]
    h2 = jnp.maximum(agg * inv + b1_ref[...] + r1_ref[...], 0.0)
    o = jnp.dot(h2, wlin_ref[...], preferred_element_type=jnp.float32)
    o = o + blin_ref[...]
    m = jnp.max(o, axis=1, keepdims=True)
    e = jnp.exp(o - m)
    out_ref[...] = e / jnp.sum(e, axis=1, keepdims=True)


def _tc_final(agg1, hists, r1, b1, wlin, blin):
    return pl.pallas_call(
        _tc_final_body,
        grid=(NBLK,),
        in_specs=[pl.BlockSpec((NC, BLK, D), lambda i: (0, i, 0)),
                  pl.BlockSpec((NC * NS, BLK, 1), lambda i: (0, i, 0)),
                  pl.BlockSpec((BLK, D), lambda i: (i, 0)),
                  pl.BlockSpec((1, D), lambda i: (0, 0)),
                  pl.BlockSpec((D, DO), lambda i: (0, 0)),
                  pl.BlockSpec((1, DO), lambda i: (0, 0))],
        out_specs=pl.BlockSpec((BLK, DO), lambda i: (i, 0)),
        out_shape=jax.ShapeDtypeStruct((N, DO), jnp.float32),
    )(agg1, hists, r1, b1, wlin, blin)


_sc_agg_hist = _make_sc_agg(True, 2)
_sc_agg = _make_sc_agg(False, 3)


def kernel(x, edge_index, Wl0, b0, Wr0, Wl1, b1, Wr1, Wlin, blin):
    src = edge_index[0]
    dst = edge_index[1]

    p0 = _tc_project(x, Wl0)
    agg0, hists = _sc_agg_hist(p0, src, dst)
    hists = hists.reshape(NC * NS, N, 1)
    p1, r1 = _tc_mid(agg0, hists, x, Wr0, b0.reshape(1, D), Wl1, Wr1)
    agg1, = _sc_agg(p1, src, dst)
    out = _tc_final(agg1, hists, r1, b1.reshape(1, D), Wlin,
                    blin.reshape(1, DO))
    return out


# speedup vs baseline: 1.9884x; 1.7262x over previous
import functools

import jax
import jax.numpy as jnp
from jax import lax
from jax.experimental import pallas as pl
from jax.experimental.pallas import tpu as pltpu
from jax.experimental.pallas import tpu_sc as plsc

N = 10000
E = 320000
D = 128
NC = 2
NS = 16

EPC = E // NC
EPT = EPC // NS
G = 128
NG = EPT // G
REM = EPT - NG * G

ROWS_PT = 624
TAIL = N - ROWS_PT * NS


def _sc_agg_body(with_hist, nbuf, *refs):
    it = iter(refs)
    tbl_h = next(it)
    src_h = next(it)
    dst_h = next(it)
    agg_h = next(it)
    hists_h = next(it) if with_hist else None
    agg_sp = next(it)
    srcs = [next(it) for _ in range(nbuf)]
    dsts = [next(it) for _ in range(nbuf)]
    srcr = next(it)
    dstr = next(it)
    rows = [next(it) for _ in range(nbuf)]
    hist = next(it) if with_hist else None
    isem = next(it)
    gsem = next(it)
    ssem = next(it)

    cid = lax.axis_index("c")
    sid = lax.axis_index("s")

    zeros16 = jnp.zeros((16,), jnp.float32)

    def zrow(i, c):
        for j in range(D // 16):
            rows[0][i, pl.ds(j * 16, 16)] = zeros16
        return c

    lax.fori_loop(0, G, zrow, 0)
    zb = sid * ROWS_PT
    for k in range(4):
        pltpu.sync_copy(rows[0], agg_sp.at[pl.ds(zb + k * 128, 128)])
    pltpu.sync_copy(rows[0].at[pl.ds(0, ROWS_PT - 512)],
                    agg_sp.at[pl.ds(zb + 512, ROWS_PT - 512)])

    @pl.when(sid == 0)
    def _():
        pltpu.sync_copy(rows[0].at[pl.ds(0, TAIL)],
                        agg_sp.at[pl.ds(ROWS_PT * NS, TAIL)])

    ones16 = jnp.full((16,), 1.0, jnp.float32)
    if with_hist:
        def zh(i, c):
            hist[pl.ds(i * 16, 16)] = zeros16
            return c

        lax.fori_loop(0, N // 16, zh, 0)

    plsc.subcore_barrier()

    tile_base = cid * EPC + sid * EPT

    def chunk(q, c):
        g0 = q * nbuf
        ids = []
        for i in range(nbuf):
            base = tile_base + (g0 + i) * G
            a = pltpu.async_copy(src_h.at[pl.ds(base, G)], srcs[i], isem)
            b = pltpu.async_copy(dst_h.at[pl.ds(base, G)], dsts[i], isem)
            ids.append((a, b))
        gds = []
        for i in range(nbuf):
            ids[i][0].wait()
            ids[i][1].wait()
            gds.append(pltpu.async_copy(tbl_h.at[srcs[i]], rows[i], gsem))
        sds = []
        for i in range(nbuf):
            gds[i].wait()
            sds.append(pltpu.async_copy(rows[i], agg_sp.at[dsts[i]], ssem,
                                        add=True))
            if with_hist:
                for j in range(G // 16):
                    v = dsts[i][pl.ds(j * 16, 16)]
                    plsc.addupdate_scatter(hist, [v], ones16)
        for i in range(nbuf):
            sds[i].wait()
        return c

    lax.fori_loop(0, NG // nbuf, chunk, 0)

    rbase = tile_base + NG * G
    ra = pltpu.async_copy(src_h.at[pl.ds(rbase, REM)], srcr, isem)
    rb = pltpu.async_copy(dst_h.at[pl.ds(rbase, REM)], dstr, isem)
    ra.wait()
    rb.wait()
    pltpu.async_copy(tbl_h.at[srcr], rows[0].at[pl.ds(0, REM)], gsem).wait()
    pltpu.async_copy(rows[0].at[pl.ds(0, REM)], agg_sp.at[dstr], ssem,
                     add=True).wait()
    if with_hist:
        for j in range(REM // 16):
            v = dstr[pl.ds(j * 16, 16)]
            plsc.addupdate_scatter(hist, [v], ones16)
        pltpu.sync_copy(hist, hists_h.at[cid * NS + sid, 0])

    plsc.subcore_barrier()

    ob = sid * ROWS_PT
    pltpu.sync_copy(agg_sp.at[pl.ds(ob, ROWS_PT)],
                    agg_h.at[cid, pl.ds(ob, ROWS_PT)])

    @pl.when(sid == 0)
    def _():
        pltpu.sync_copy(agg_sp.at[pl.ds(ROWS_PT * NS, TAIL)],
                        agg_h.at[cid, pl.ds(ROWS_PT * NS, TAIL)])


def _make_sc_agg(with_hist, nbuf):
    assert NG % nbuf == 0
    out_type = [jax.ShapeDtypeStruct((NC, N, D), jnp.float32)]
    if with_hist:
        out_type.append(jax.ShapeDtypeStruct((NC * NS, 1, N), jnp.float32))
    scratch = [pltpu.VMEM_SHARED((N, D), jnp.float32)]
    scratch += [pltpu.VMEM((G,), jnp.int32) for _ in range(2 * nbuf)]
    scratch += [pltpu.VMEM((REM,), jnp.int32)] * 2
    scratch += [pltpu.VMEM((G, D), jnp.float32) for _ in range(nbuf)]
    if with_hist:
        scratch.append(pltpu.VMEM((N,), jnp.float32))
    scratch += [pltpu.SemaphoreType.DMA] * 3
    return pl.kernel(
        functools.partial(_sc_agg_body, with_hist, nbuf),
        out_type=tuple(out_type),
        mesh=plsc.VectorSubcoreMesh(core_axis_name="c", subcore_axis_name="s"),
        scratch_types=scratch,
        compiler_params=pltpu.CompilerParams(needs_layout_passes=False),
    )


def _inv_deg_blk(h_ref):
    deg = jnp.sum(jnp.transpose(h_ref[:, 0, :]), axis=1, keepdims=True)
    return 1.0 / jnp.maximum(deg, 1.0)


def _tc_mid_body(agg_ref, h_ref, x_ref, wl0_ref, b0_ref, wr0_ref, h_out):
    inv = _inv_deg_blk(h_ref)
    mean = (agg_ref[0] + agg_ref[1]) * inv
    h_out[...] = jnp.maximum(
        jnp.dot(mean, wl0_ref[...], preferred_element_type=jnp.float32)
        + b0_ref[...]
        + jnp.dot(x_ref[...], wr0_ref[...], preferred_element_type=jnp.float32),
        0.0)


def _tc_mid(agg0, hists, x, wl0, b0, wr0):
    return pl.pallas_call(
        _tc_mid_body,
        out_shape=jax.ShapeDtypeStruct((N, D), jnp.float32),
    )(agg0, hists, x, wl0, b0, wr0)


DO = 64


def _tc_final_body(agg_ref, h_ref, x1_ref, wl1_ref, b1_ref, wr1_ref,
                   wlin_ref, blin_ref, out_ref):
    inv = _inv_deg_blk(h_ref)
    mean = (agg_ref[0] + agg_ref[1]) * inv
    h2 = jnp.maximum(
        jnp.dot(mean, wl1_ref[...], preferred_element_type=jnp.float32)
        + b1_ref[...]
        + jnp.dot(x1_ref[...], wr1_ref[...],
                  preferred_element_type=jnp.float32),
        0.0)
    o = jnp.dot(h2, wlin_ref[...], preferred_element_type=jnp.float32)
    o = o + blin_ref[...]
    m = jnp.max(o, axis=1, keepdims=True)
    e = jnp.exp(o - m)
    out_ref[...] = e / jnp.sum(e, axis=1, keepdims=True)


def _tc_final(agg1, hists, h, wl1, b1, wr1, wlin, blin):
    return pl.pallas_call(
        _tc_final_body,
        out_shape=jax.ShapeDtypeStruct((N, DO), jnp.float32),
    )(agg1, hists, h, wl1, b1, wr1, wlin, blin)


_sc_agg_hist = _make_sc_agg(True, 2)
_sc_agg = _make_sc_agg(False, 3)


def kernel(x, edge_index, Wl0, b0, Wr0, Wl1, b1, Wr1, Wlin, blin):
    src = edge_index[0]
    dst = edge_index[1]

    agg0, hists = _sc_agg_hist(x, src, dst)
    h = _tc_mid(agg0, hists, x, Wl0, b0.reshape(1, D), Wr0)
    agg1, = _sc_agg(h, src, dst)
    out = _tc_final(agg1, hists, h, Wl1, b1.reshape(1, D), Wr1, Wlin,
                    blin.reshape(1, DO))
    return out


# speedup vs baseline: 2.0914x; 1.0518x over previous
import functools

import jax
import jax.numpy as jnp
from jax import lax
from jax.experimental import pallas as pl
from jax.experimental.pallas import tpu as pltpu
from jax.experimental.pallas import tpu_sc as plsc

N = 10000
E = 320000
D = 128
NC = 2
NS = 16

EPC = E // NC
G = 128
EPB = 9984
NGRP = EPB // G
EXTRA = (EPC - NS * EPB) // G

ROWS_PT = 624
TAIL = N - ROWS_PT * NS


def _sc_agg_body(with_hist, nbuf, *refs):
    it = iter(refs)
    tbl_h = next(it)
    ei_h = next(it)
    agg_h = next(it)
    hists_h = next(it) if with_hist else None
    agg_sp = next(it)
    idxs = [next(it) for _ in range(nbuf)]
    rows = [next(it) for _ in range(nbuf)]
    hist = next(it) if with_hist else None
    isem = next(it)
    gsem = next(it)
    ssem = next(it)

    cid = lax.axis_index("c")
    sid = lax.axis_index("s")

    zeros16 = jnp.zeros((16,), jnp.float32)

    def zrow(i, c):
        for j in range(D // 16):
            rows[0][i, pl.ds(j * 16, 16)] = zeros16
        return c

    lax.fori_loop(0, G, zrow, 0)
    zb = sid * ROWS_PT
    for k in range(4):
        pltpu.sync_copy(rows[0], agg_sp.at[pl.ds(zb + k * G, G)])
    pltpu.sync_copy(rows[0].at[pl.ds(0, ROWS_PT - 512)],
                    agg_sp.at[pl.ds(zb + 512, ROWS_PT - 512)])

    @pl.when(sid == 0)
    def _():
        pltpu.sync_copy(rows[0].at[pl.ds(0, TAIL)],
                        agg_sp.at[pl.ds(ROWS_PT * NS, TAIL)])

    ones16 = jnp.full((16,), 1.0, jnp.float32)
    if with_hist:
        def zh(i, c):
            hist[pl.ds(i * 16, 16)] = zeros16
            return c

        lax.fori_loop(0, N // 16, zh, 0)

    plsc.subcore_barrier()

    tile_base = cid * EPC + sid * EPB

    def hist_update(ibuf):
        for j in range(G // 16):
            v = ibuf[1, pl.ds(j * 16, 16)]
            plsc.addupdate_scatter(hist, [v], ones16)

    def run_groups(first, count):
        assert count <= nbuf
        ids = []
        for i in range(count):
            base = tile_base + (first + i) * G
            ids.append(pltpu.async_copy(ei_h.at[:, pl.ds(base, G)],
                                        idxs[i % nbuf], isem))
        gds = []
        for i in range(count):
            ids[i].wait()
            gds.append(pltpu.async_copy(tbl_h.at[idxs[i % nbuf].at[0]],
                                        rows[i % nbuf], gsem))
        sds = []
        for i in range(count):
            gds[i].wait()
            sds.append(pltpu.async_copy(rows[i % nbuf],
                                        agg_sp.at[idxs[i % nbuf].at[1]],
                                        ssem, add=True))
            if with_hist:
                hist_update(idxs[i % nbuf])
        for i in range(count):
            sds[i].wait()

    def chunk(q, c):
        run_groups(q * nbuf, nbuf)
        return c

    lax.fori_loop(0, NGRP // nbuf, chunk, 0)

    @pl.when(sid == NS - 1)
    def _():
        run_groups(NGRP, EXTRA)

    if with_hist:
        pltpu.sync_copy(hist, hists_h.at[cid * NS + sid, 0])

    plsc.subcore_barrier()

    ob = sid * ROWS_PT
    pltpu.sync_copy(agg_sp.at[pl.ds(ob, ROWS_PT)],
                    agg_h.at[cid, pl.ds(ob, ROWS_PT)])

    @pl.when(sid == 0)
    def _():
        pltpu.sync_copy(agg_sp.at[pl.ds(ROWS_PT * NS, TAIL)],
                        agg_h.at[cid, pl.ds(ROWS_PT * NS, TAIL)])


def _make_sc_agg(with_hist, nbuf):
    assert NGRP % nbuf == 0 and EXTRA <= nbuf
    out_type = [jax.ShapeDtypeStruct((NC, N, D), jnp.float32)]
    if with_hist:
        out_type.append(jax.ShapeDtypeStruct((NC * NS, 1, N), jnp.float32))
    scratch = [pltpu.VMEM_SHARED((N, D), jnp.float32)]
    scratch += [pltpu.VMEM((2, G), jnp.int32) for _ in range(nbuf)]
    scratch += [pltpu.VMEM((G, D), jnp.float32) for _ in range(nbuf)]
    if with_hist:
        scratch.append(pltpu.VMEM((N,), jnp.float32))
    scratch += [pltpu.SemaphoreType.DMA] * 3
    return pl.kernel(
        functools.partial(_sc_agg_body, with_hist, nbuf),
        out_type=tuple(out_type),
        mesh=plsc.VectorSubcoreMesh(core_axis_name="c", subcore_axis_name="s"),
        scratch_types=scratch,
        compiler_params=pltpu.CompilerParams(needs_layout_passes=False),
    )


def _inv_deg_blk(h_ref):
    deg = jnp.sum(jnp.transpose(h_ref[:, 0, :]), axis=1, keepdims=True)
    return 1.0 / jnp.maximum(deg, 1.0)


def _tc_mid_body(agg_ref, h_ref, x_ref, wl0_ref, b0_ref, wr0_ref, h_out):
    inv = _inv_deg_blk(h_ref)
    mean = (agg_ref[0] + agg_ref[1]) * inv
    h_out[...] = jnp.maximum(
        jnp.dot(mean, wl0_ref[...], preferred_element_type=jnp.float32)
        + b0_ref[...]
        + jnp.dot(x_ref[...], wr0_ref[...], preferred_element_type=jnp.float32),
        0.0)


def _tc_mid(agg0, hists, x, wl0, b0, wr0):
    return pl.pallas_call(
        _tc_mid_body,
        out_shape=jax.ShapeDtypeStruct((N, D), jnp.float32),
    )(agg0, hists, x, wl0, b0, wr0)


DO = 64


def _tc_final_body(agg_ref, h_ref, x1_ref, wl1_ref, b1_ref, wr1_ref,
                   wlin_ref, blin_ref, out_ref):
    inv = _inv_deg_blk(h_ref)
    mean = (agg_ref[0] + agg_ref[1]) * inv
    h2 = jnp.maximum(
        jnp.dot(mean, wl1_ref[...], preferred_element_type=jnp.float32)
        + b1_ref[...]
        + jnp.dot(x1_ref[...], wr1_ref[...],
                  preferred_element_type=jnp.float32),
        0.0)
    o = jnp.dot(h2, wlin_ref[...], preferred_element_type=jnp.float32)
    o = o + blin_ref[...]
    m = jnp.max(o, axis=1, keepdims=True)
    e = jnp.exp(o - m)
    out_ref[...] = e / jnp.sum(e, axis=1, keepdims=True)


def _tc_final(agg1, hists, h, wl1, b1, wr1, wlin, blin):
    return pl.pallas_call(
        _tc_final_body,
        out_shape=jax.ShapeDtypeStruct((N, DO), jnp.float32),
    )(agg1, hists, h, wl1, b1, wr1, wlin, blin)


_sc_agg_hist = _make_sc_agg(True, 2)
_sc_agg = _make_sc_agg(False, 3)


def kernel(x, edge_index, Wl0, b0, Wr0, Wl1, b1, Wr1, Wlin, blin):
    agg0, hists = _sc_agg_hist(x, edge_index)
    h = _tc_mid(agg0, hists, x, Wl0, b0.reshape(1, D), Wr0)
    agg1, = _sc_agg(h, edge_index)
    out = _tc_final(agg1, hists, h, Wl1, b1.reshape(1, D), Wr1, Wlin,
                    blin.reshape(1, DO))
    return out
